# baseline (speedup 1.0000x reference)
import functools

import jax
import jax.numpy as jnp
from jax import lax
from jax.experimental import pallas as pl
from jax.experimental.pallas import tpu as pltpu

_N = 8192
_D = 768
_K = 10
_CH = 1024
_G = _N // _CH

_SAMPLE_IDX = (3302, 333, 4909, 3563, 708, 5151, 8056, 4474, 3236, 4658)


def _body(e_ref, wq_ref, wk_ref, bq_ref, bk_ref, *rest):
    samp_refs = rest[:_K]
    o_ref = rest[_K]
    ebf_ref, m_ref, nk_ref, wqb_ref, wkb_ref = rest[_K + 1 :]
    f32, bf16 = jnp.float32, jnp.bfloat16
    i = pl.program_id(0)

    @pl.when(i == 0)
    def _prologue():
        wqb_ref[...] = wq_ref[...].astype(bf16)
        wkb_ref[...] = wk_ref[...].astype(bf16)
        g = jnp.concatenate(
            [r[(row % 8) : (row % 8) + 1, :] for r, row in zip(samp_refs, _SAMPLE_IDX)],
            axis=0,
        ).astype(bf16)
        nk = lax.dot_general(g, wkb_ref[...], (((1,), (1,)), ((), ())),
                             preferred_element_type=f32) + bk_ref[...]
        nk_ref[...] = nk.astype(bf16)

    eb = e_ref[...].astype(bf16)
    ebf_ref[pl.ds(i * _CH, _CH), :] = eb
    q = lax.dot_general(eb, wqb_ref[...], (((1,), (1,)), ((), ())),
                        preferred_element_type=f32) + bq_ref[...]
    s = lax.dot_general(q.astype(bf16), nk_ref[...], (((1,), (1,)), ((), ())),
                        preferred_element_type=f32)
    m_ref[pl.ds(i * (_CH // 128), _CH // 128), :] = jnp.max(s, axis=1).reshape(
        _CH // 128, 128
    )

    @pl.when(i == _G - 1)
    def _finale():
        m = m_ref[...]
        iota = (
            lax.broadcasted_iota(jnp.int32, (64, 128), 0) * 128
            + lax.broadcasted_iota(jnp.int32, (64, 128), 1)
        )
        rowi = lax.broadcasted_iota(jnp.int32, (_K, 1), 0)
        idxvec = jnp.zeros((_K, 1), jnp.int32)
        for k in range(_K):
            v = jnp.max(m)
            idx = jnp.min(jnp.where(m == v, iota, jnp.int32(_N)))
            m = jnp.where(iota == idx, jnp.float32(-jnp.inf), m)
            idxvec = jnp.where(rowi == k, idx, idxvec)

        onehot = (
            lax.broadcasted_iota(jnp.int32, (_K, _N), 1) == idxvec
        ).astype(bf16)
        ebf = ebf_ref[...]
        g2 = lax.dot_general(onehot, ebf, (((1,), (0,)), ((), ())),
                             preferred_element_type=f32).astype(bf16)
        qr = lax.dot_general(g2, wqb_ref[...], (((1,), (1,)), ((), ())),
                             preferred_element_type=f32) + bq_ref[...]
        qrb = qr.astype(bf16)
        r = lax.dot_general(qrb, wkb_ref[...], (((1,), (0,)), ((), ())),
                            preferred_element_type=f32)
        c2 = lax.dot_general(bk_ref[...], qrb, (((1,), (1,)), ((), ())),
                             preferred_element_type=f32)

        t = lax.dot_general(ebf, r.astype(bf16), (((1,), (1,)), ((), ())),
                            preferred_element_type=f32) + c2
        pooled = jnp.max(t, axis=1, keepdims=True)
        o_ref[...] = jnp.sum(pooled * ebf.astype(f32), axis=0, keepdims=True)


@jax.jit
def _run(embed_matrix, Wq, bq, Wk, bk):
    samp_specs = [
        pl.BlockSpec((8, _D), lambda i, b=row // 8: (b, 0)) for row in _SAMPLE_IDX
    ]
    return pl.pallas_call(
        _body,
        grid=(_G,),
        in_specs=[
            pl.BlockSpec((_CH, _D), lambda i: (i, 0)),
            pl.BlockSpec((_D, _D), lambda i: (0, 0)),
            pl.BlockSpec((_D, _D), lambda i: (0, 0)),
            pl.BlockSpec((1, _D), lambda i: (0, 0)),
            pl.BlockSpec((1, _D), lambda i: (0, 0)),
            *samp_specs,
        ],
        out_specs=pl.BlockSpec((1, _D), lambda i: (0, 0)),
        out_shape=jax.ShapeDtypeStruct((1, _D), jnp.float32),
        scratch_shapes=[
            pltpu.VMEM((_N, _D), jnp.bfloat16),
            pltpu.VMEM((64, 128), jnp.float32),
            pltpu.VMEM((_K, _D), jnp.bfloat16),
            pltpu.VMEM((_D, _D), jnp.bfloat16),
            pltpu.VMEM((_D, _D), jnp.bfloat16),
        ],
    )(
        embed_matrix,
        Wq,
        Wk,
        bq.reshape(1, _D),
        bk.reshape(1, _D),
        *([embed_matrix] * _K),
    )


def kernel(embed_matrix, Wq, bq, Wk, bk):
    return _run(embed_matrix, Wq, bq, Wk, bk)
